# baseline (speedup 1.0000x reference)
import functools

import jax
import jax.numpy as jnp
from jax import lax
from jax.experimental import pallas as pl
from jax.experimental.pallas import tpu as pltpu
from jax.experimental.pallas import tpu_sc as plsc

NC, NS, L = 2, 16, 16
NW = NC * NS
B, S, H = 4, 2048, 768
TOK = B * S
TPW = TOK // NW
C = 32
NJ = H // L
EPS = 1e-12


def _body(ids, tts, word, pos, typ, gamma, beta, out,
          idxc, ttc, wbuf, tbuf, pbuf, obuf, g_v, b_v, sem):
    wid = lax.axis_index("s") * NC + lax.axis_index("c")
    base = wid * TPW
    s_base = lax.rem(base, S)

    pltpu.sync_copy(gamma, g_v)
    pltpu.sync_copy(beta, b_v)

    def chunk_body(c, carry):
        tok0 = base + c * C
        s0 = s_base + c * C
        pltpu.sync_copy(ids.at[pl.ds(tok0, C)], idxc)
        pltpu.sync_copy(tts.at[pl.ds(tok0, C)], ttc)
        pltpu.async_copy(word.at[idxc], wbuf, sem).wait()
        pltpu.async_copy(typ.at[ttc], tbuf, sem).wait()
        pltpu.sync_copy(pos.at[pl.ds(s0, C)], pbuf)

        def tok_body(t, carry2):
            acc_s = jnp.zeros((L,), jnp.float32)
            acc_q = jnp.zeros((L,), jnp.float32)
            for j in range(NJ):
                sl = pl.ds(j * L, L)
                x = wbuf[t, sl] + pbuf[t, sl] + tbuf[t, sl]
                acc_s = acc_s + x
                acc_q = acc_q + x * x
            mean = jnp.sum(acc_s) * (1.0 / H)
            var = jnp.sum(acc_q) * (1.0 / H) - mean * mean
            v = jnp.full((L,), var + EPS, jnp.float32)
            i = lax.bitcast_convert_type(v, jnp.int32)
            i = 0x5F3759DF - lax.shift_right_logical(i, 1)
            r = lax.bitcast_convert_type(i, jnp.float32)
            for _ in range(4):
                r = r * (1.5 - 0.5 * v * r * r)
            mv = jnp.full((L,), mean, jnp.float32)
            for j in range(NJ):
                sl = pl.ds(j * L, L)
                x = wbuf[t, sl] + pbuf[t, sl] + tbuf[t, sl]
                obuf[t, sl] = (x - mv) * r * g_v[sl] + b_v[sl]
            return carry2

        lax.fori_loop(0, C, tok_body, 0)
        pltpu.sync_copy(obuf, out.at[pl.ds(tok0, C)])
        return carry

    lax.fori_loop(0, TPW // C, chunk_body, 0)


_sc_embed = functools.partial(
    pl.kernel,
    mesh=plsc.VectorSubcoreMesh(core_axis_name="c", subcore_axis_name="s"),
    out_type=jax.ShapeDtypeStruct((TOK, H), jnp.float32),
    scratch_types=[
        pltpu.VMEM((C,), jnp.int32),
        pltpu.VMEM((C,), jnp.int32),
        pltpu.VMEM((C, H), jnp.float32),
        pltpu.VMEM((C, H), jnp.float32),
        pltpu.VMEM((C, H), jnp.float32),
        pltpu.VMEM((C, H), jnp.float32),
        pltpu.VMEM((H,), jnp.float32),
        pltpu.VMEM((H,), jnp.float32),
        pltpu.SemaphoreType.DMA,
    ],
    compiler_params=pltpu.CompilerParams(needs_layout_passes=False),
)(_body)


@jax.jit
def kernel(input_ids, token_type_ids, word_emb, pos_emb, type_emb, gamma, beta):
    ids = input_ids.reshape(-1).astype(jnp.int32)
    tts = token_type_ids.reshape(-1).astype(jnp.int32)
    out = _sc_embed(ids, tts, word_emb, pos_emb, type_emb, gamma, beta)
    return out.reshape(B, S, H)


# speedup vs baseline: 1.1573x; 1.1573x over previous
import functools

import jax
import jax.numpy as jnp
from jax import lax
from jax.experimental import pallas as pl
from jax.experimental.pallas import tpu as pltpu
from jax.experimental.pallas import tpu_sc as plsc

NC, NS, L = 2, 16, 16
NW = NC * NS
B, S, H = 4, 2048, 768
TOK = B * S
TPW = TOK // NW
C = 16
NCH = TPW // C
NJ = H // L
EPS = 1e-12


def _body(ids, tts, word, pos, typ, gamma, beta, out,
          idxc, ttc, wbuf, pbuf, tbuf, obuf, g_v, b_v, sem_g, sem_o):
    wid = lax.axis_index("s") * NC + lax.axis_index("c")
    base = wid * TPW
    s_base = lax.rem(base, S)

    pltpu.sync_copy(gamma, g_v)
    pltpu.sync_copy(beta, b_v)

    def issue(cc, b):
        tok0 = base + cc * C
        s0 = s_base + cc * C
        pltpu.sync_copy(ids.at[pl.ds(tok0, C)], idxc[b])
        pltpu.sync_copy(tts.at[pl.ds(tok0, C)], ttc[b])
        pltpu.async_copy(word.at[idxc[b]], wbuf[b], sem_g[b])
        pltpu.async_copy(typ.at[ttc[b]], tbuf[b], sem_g[b])
        pltpu.async_copy(pos.at[pl.ds(s0, C)], pbuf[b], sem_g[b])

    def wait_gathers(b):
        pltpu.make_async_copy(word.at[idxc[b]], wbuf[b], sem_g[b]).wait()
        pltpu.make_async_copy(typ.at[ttc[b]], tbuf[b], sem_g[b]).wait()
        pltpu.make_async_copy(pos.at[pl.ds(0, C)], pbuf[b], sem_g[b]).wait()

    def wait_scatter(cc, b):
        tok0 = base + cc * C
        pltpu.make_async_copy(obuf[b], out.at[pl.ds(tok0, C)], sem_o[b]).wait()

    def compute(b):
        w, o = wbuf[b], obuf[b]

        def tok_body(t, carry2):
            accs = [jnp.zeros((L,), jnp.float32) for _ in range(4)]
            accq = [jnp.zeros((L,), jnp.float32) for _ in range(4)]
            for j in range(NJ):
                sl = pl.ds(j * L, L)
                x = w[t, sl] + pbuf[b][t, sl] + tbuf[b][t, sl]
                w[t, sl] = x
                accs[j % 4] = accs[j % 4] + x
                accq[j % 4] = accq[j % 4] + x * x
            acc_s = (accs[0] + accs[1]) + (accs[2] + accs[3])
            acc_q = (accq[0] + accq[1]) + (accq[2] + accq[3])
            mean = jnp.sum(acc_s) * (1.0 / H)
            var = jnp.sum(acc_q) * (1.0 / H) - mean * mean
            v = jnp.full((L,), var + EPS, jnp.float32)
            i = lax.bitcast_convert_type(v, jnp.int32)
            i = 0x5F3759DF - lax.shift_right_logical(i, 1)
            r = lax.bitcast_convert_type(i, jnp.float32)
            for _ in range(4):
                r = r * (1.5 - 0.5 * v * r * r)
            mv = jnp.full((L,), mean, jnp.float32)
            for j in range(NJ):
                sl = pl.ds(j * L, L)
                o[t, sl] = (w[t, sl] - mv) * r * g_v[sl] + b_v[sl]
            return carry2

        lax.fori_loop(0, C, tok_body, 0)

    issue(0, 0)
    issue(1, 1)

    def pair_body(i, carry):
        for b in range(2):
            cc = 2 * i + b
            wait_gathers(b)

            @pl.when(cc >= 2)
            def _():
                wait_scatter(cc - 2, b)

            compute(b)
            tok0 = base + cc * C
            pltpu.async_copy(obuf[b], out.at[pl.ds(tok0, C)], sem_o[b])

            @pl.when(cc + 2 < NCH)
            def _():
                issue(cc + 2, b)
        return carry

    lax.fori_loop(0, NCH // 2, pair_body, 0)
    wait_scatter(NCH - 2, 0)
    wait_scatter(NCH - 1, 1)


_sc_embed = functools.partial(
    pl.kernel,
    mesh=plsc.VectorSubcoreMesh(core_axis_name="c", subcore_axis_name="s"),
    out_type=jax.ShapeDtypeStruct((TOK, H), jnp.float32),
    scratch_types=[
        [pltpu.VMEM((C,), jnp.int32) for _ in range(2)],
        [pltpu.VMEM((C,), jnp.int32) for _ in range(2)],
        [pltpu.VMEM((C, H), jnp.float32) for _ in range(2)],
        [pltpu.VMEM((C, H), jnp.float32) for _ in range(2)],
        [pltpu.VMEM((C, H), jnp.float32) for _ in range(2)],
        [pltpu.VMEM((C, H), jnp.float32) for _ in range(2)],
        pltpu.VMEM((H,), jnp.float32),
        pltpu.VMEM((H,), jnp.float32),
        [pltpu.SemaphoreType.DMA for _ in range(2)],
        [pltpu.SemaphoreType.DMA for _ in range(2)],
    ],
    compiler_params=pltpu.CompilerParams(needs_layout_passes=False),
)(_body)


@jax.jit
def kernel(input_ids, token_type_ids, word_emb, pos_emb, type_emb, gamma, beta):
    ids = input_ids.reshape(-1).astype(jnp.int32)
    tts = token_type_ids.reshape(-1).astype(jnp.int32)
    out = _sc_embed(ids, tts, word_emb, pos_emb, type_emb, gamma, beta)
    return out.reshape(B, S, H)


# speedup vs baseline: 1.1976x; 1.0348x over previous
import functools

import jax
import jax.numpy as jnp
from jax import lax
from jax.experimental import pallas as pl
from jax.experimental.pallas import tpu as pltpu
from jax.experimental.pallas import tpu_sc as plsc

NC, NS, L = 2, 16, 16
NW = NC * NS
B, S, H = 4, 2048, 768
TOK = B * S
TPW = TOK // NW
C = 16
NCH = TPW // C
NJ = H // L
EPS = 1e-12


def _body(ids, tts, word, pos, typ, gamma, beta, out,
          idxc, ttc, wbuf, pbuf, tbuf, xbuf, obuf, g_v, b_v, sem_g, sem_o):
    wid = lax.axis_index("s") * NC + lax.axis_index("c")
    base = wid * TPW
    s_base = lax.rem(base, S)

    pltpu.sync_copy(gamma, g_v)
    pltpu.sync_copy(beta, b_v)

    def issue(cc, b):
        tok0 = base + cc * C
        s0 = s_base + cc * C
        pltpu.sync_copy(ids.at[pl.ds(tok0, C)], idxc[b])
        pltpu.sync_copy(tts.at[pl.ds(tok0, C)], ttc[b])
        pltpu.async_copy(word.at[idxc[b]], wbuf[b], sem_g[b])
        pltpu.async_copy(typ.at[ttc[b]], tbuf[b], sem_g[b])
        pltpu.async_copy(pos.at[pl.ds(s0, C)], pbuf[b], sem_g[b])

    def wait_gathers(b):
        pltpu.make_async_copy(word.at[idxc[b]], wbuf[b], sem_g[b]).wait()
        pltpu.make_async_copy(typ.at[ttc[b]], tbuf[b], sem_g[b]).wait()
        pltpu.make_async_copy(pos.at[pl.ds(0, C)], pbuf[b], sem_g[b]).wait()

    def wait_scatter(cc, b):
        tok0 = base + cc * C
        pltpu.make_async_copy(obuf[b], out.at[pl.ds(tok0, C)], sem_o[b]).wait()

    def compute(b):
        w, p, ty, x_, o = wbuf[b], pbuf[b], tbuf[b], xbuf[b], obuf[b]
        z = jnp.zeros((L,), jnp.float32)

        def tok_body(t, carry2):
            @plsc.parallel_loop(0, NJ, step=4, carry=(z,) * 8)
            def p1(j, c):
                acc = list(c)
                for u in range(4):
                    sl = pl.ds((j + u) * L, L)
                    x = w[t, sl] + p[t, sl] + ty[t, sl]
                    x_[t, sl] = x
                    acc[u] = acc[u] + x
                    acc[4 + u] = acc[4 + u] + x * x
                return tuple(acc)

            acc_s = (p1[0] + p1[1]) + (p1[2] + p1[3])
            acc_q = (p1[4] + p1[5]) + (p1[6] + p1[7])
            mean = jnp.sum(acc_s) * (1.0 / H)
            var = jnp.sum(acc_q) * (1.0 / H) - mean * mean
            v = jnp.full((L,), var + EPS, jnp.float32)
            i = lax.bitcast_convert_type(v, jnp.int32)
            i = 0x5F3759DF - lax.shift_right_logical(i, 1)
            r = lax.bitcast_convert_type(i, jnp.float32)
            for _ in range(4):
                r = r * (1.5 - 0.5 * v * r * r)
            mv = jnp.full((L,), mean, jnp.float32)

            @plsc.parallel_loop(0, NJ, step=4)
            def p2(j):
                for u in range(4):
                    sl = pl.ds((j + u) * L, L)
                    o[t, sl] = (x_[t, sl] - mv) * r * g_v[sl] + b_v[sl]

            return carry2

        lax.fori_loop(0, C, tok_body, 0)

    issue(0, 0)
    issue(1, 1)

    def pair_body(i, carry):
        for b in range(2):
            cc = 2 * i + b
            wait_gathers(b)

            @pl.when(cc >= 2)
            def _():
                wait_scatter(cc - 2, b)

            compute(b)
            tok0 = base + cc * C
            pltpu.async_copy(obuf[b], out.at[pl.ds(tok0, C)], sem_o[b])

            @pl.when(cc + 2 < NCH)
            def _():
                issue(cc + 2, b)
        return carry

    lax.fori_loop(0, NCH // 2, pair_body, 0)
    wait_scatter(NCH - 2, 0)
    wait_scatter(NCH - 1, 1)


_sc_embed = functools.partial(
    pl.kernel,
    mesh=plsc.VectorSubcoreMesh(core_axis_name="c", subcore_axis_name="s"),
    out_type=jax.ShapeDtypeStruct((TOK, H), jnp.float32),
    scratch_types=[
        [pltpu.VMEM((C,), jnp.int32) for _ in range(2)],
        [pltpu.VMEM((C,), jnp.int32) for _ in range(2)],
        [pltpu.VMEM((C, H), jnp.float32) for _ in range(2)],
        [pltpu.VMEM((C, H), jnp.float32) for _ in range(2)],
        [pltpu.VMEM((C, H), jnp.float32) for _ in range(2)],
        [pltpu.VMEM((C, H), jnp.float32) for _ in range(2)],
        [pltpu.VMEM((C, H), jnp.float32) for _ in range(2)],
        pltpu.VMEM((H,), jnp.float32),
        pltpu.VMEM((H,), jnp.float32),
        [pltpu.SemaphoreType.DMA for _ in range(2)],
        [pltpu.SemaphoreType.DMA for _ in range(2)],
    ],
    compiler_params=pltpu.CompilerParams(needs_layout_passes=False),
)(_body)


@jax.jit
def kernel(input_ids, token_type_ids, word_emb, pos_emb, type_emb, gamma, beta):
    ids = input_ids.reshape(-1).astype(jnp.int32)
    tts = token_type_ids.reshape(-1).astype(jnp.int32)
    out = _sc_embed(ids, tts, word_emb, pos_emb, type_emb, gamma, beta)
    return out.reshape(B, S, H)


# speedup vs baseline: 3.0203x; 2.5220x over previous
import functools

import jax
import jax.numpy as jnp
from jax import lax
from jax.experimental import pallas as pl
from jax.experimental.pallas import tpu as pltpu
from jax.experimental.pallas import tpu_sc as plsc

NC, NS, L = 2, 16, 16
NW = NC * NS
B, S, H = 4, 2048, 768
TOK = B * S
TPW = TOK // NW
C = 16
NCH = TPW // C
NJ = H // L
EPS = 1e-12


_DNUMS = lax.GatherDimensionNumbers(
    offset_dims=(), collapsed_slice_dims=(0,), start_index_map=(0,))


def _lane_broadcast(vec, t):
    idxv = jnp.full((L,), t, jnp.int32)
    return lax.gather(vec, idxv[:, None], _DNUMS, slice_sizes=(1,),
                      mode=lax.GatherScatterMode.PROMISE_IN_BOUNDS)


def _body(ids, tts, word, pos, typ, gamma, beta, out,
          idx_v, tt_v, tybuf, wbuf, pbuf, xbuf, obuf, g_v, b_v,
          sem_g, sem_o):
    wid = lax.axis_index("s") * NC + lax.axis_index("c")
    base = wid * TPW
    s_base = lax.rem(base, S)

    pltpu.sync_copy(gamma, g_v)
    pltpu.sync_copy(beta, b_v)
    pltpu.sync_copy(typ, tybuf)
    pltpu.sync_copy(ids.at[pl.ds(base, TPW)], idx_v)
    pltpu.sync_copy(tts.at[pl.ds(base, TPW)], tt_v)

    def issue(cc, b):
        s0 = s_base + cc * C
        pltpu.async_copy(word.at[idx_v.at[pl.ds(cc * C, C)]], wbuf[b], sem_g[b])
        pltpu.async_copy(pos.at[pl.ds(s0, C)], pbuf[b], sem_g[b])

    def wait_gathers(b):
        pltpu.make_async_copy(word.at[idx_v.at[pl.ds(0, C)]], wbuf[b],
                              sem_g[b]).wait()
        pltpu.make_async_copy(pos.at[pl.ds(0, C)], pbuf[b], sem_g[b]).wait()

    def wait_scatter(cc, b):
        tok0 = base + cc * C
        pltpu.make_async_copy(obuf[b], out.at[pl.ds(tok0, C)], sem_o[b]).wait()

    def compute(cc, b):
        w, p, x_, o = wbuf[b], pbuf[b], xbuf[b], obuf[b]
        z = jnp.zeros((L,), jnp.float32)

        ttv = tt_v[pl.ds(cc * C, L)]

        def tok_body(t, carry2):
            mask = _lane_broadcast(ttv, t) > 0

            @plsc.parallel_loop(0, NJ, step=4, carry=(z,) * 8)
            def p1(j, c):
                acc = list(c)
                for u in range(4):
                    sl = pl.ds((j + u) * L, L)
                    ty = jnp.where(mask, tybuf[1, sl], tybuf[0, sl])
                    x = w[t, sl] + p[t, sl] + ty
                    x_[t, sl] = x
                    acc[u] = acc[u] + x
                    acc[4 + u] = acc[4 + u] + x * x
                return tuple(acc)

            acc_s = (p1[0] + p1[1]) + (p1[2] + p1[3])
            acc_q = (p1[4] + p1[5]) + (p1[6] + p1[7])
            mean = jnp.sum(acc_s) * (1.0 / H)
            var = jnp.sum(acc_q) * (1.0 / H) - mean * mean
            v = jnp.full((L,), var + EPS, jnp.float32)
            i = lax.bitcast_convert_type(v, jnp.int32)
            i = 0x5F3759DF - lax.shift_right_logical(i, 1)
            r = lax.bitcast_convert_type(i, jnp.float32)
            for _ in range(4):
                r = r * (1.5 - 0.5 * v * r * r)
            mv = jnp.full((L,), mean, jnp.float32)

            @plsc.parallel_loop(0, NJ, step=4)
            def p2(j):
                for u in range(4):
                    sl = pl.ds((j + u) * L, L)
                    o[t, sl] = (x_[t, sl] - mv) * r * g_v[sl] + b_v[sl]

            return carry2

        lax.fori_loop(0, C, tok_body, 0)

    issue(0, 0)
    issue(1, 1)

    def pair_body(i, carry):
        for b in range(2):
            cc = 2 * i + b
            wait_gathers(b)

            @pl.when(cc >= 2)
            def _():
                wait_scatter(cc - 2, b)

            compute(cc, b)
            tok0 = base + cc * C
            pltpu.async_copy(obuf[b], out.at[pl.ds(tok0, C)], sem_o[b])

            @pl.when(cc + 2 < NCH)
            def _():
                issue(cc + 2, b)
        return carry

    lax.fori_loop(0, NCH // 2, pair_body, 0)
    wait_scatter(NCH - 2, 0)
    wait_scatter(NCH - 1, 1)


_sc_embed = functools.partial(
    pl.kernel,
    mesh=plsc.VectorSubcoreMesh(core_axis_name="c", subcore_axis_name="s"),
    out_type=jax.ShapeDtypeStruct((TOK, H), jnp.float32),
    scratch_types=[
        pltpu.VMEM((TPW,), jnp.int32),
        pltpu.VMEM((TPW,), jnp.int32),
        pltpu.VMEM((2, H), jnp.float32),
        [pltpu.VMEM((C, H), jnp.float32) for _ in range(2)],
        [pltpu.VMEM((C, H), jnp.float32) for _ in range(2)],
        [pltpu.VMEM((C, H), jnp.float32) for _ in range(2)],
        [pltpu.VMEM((C, H), jnp.float32) for _ in range(2)],
        pltpu.VMEM((H,), jnp.float32),
        pltpu.VMEM((H,), jnp.float32),
        [pltpu.SemaphoreType.DMA for _ in range(2)],
        [pltpu.SemaphoreType.DMA for _ in range(2)],
    ],
    compiler_params=pltpu.CompilerParams(needs_layout_passes=False),
)(_body)


@jax.jit
def kernel(input_ids, token_type_ids, word_emb, pos_emb, type_emb, gamma, beta):
    ids = input_ids.reshape(-1).astype(jnp.int32)
    tts = token_type_ids.reshape(-1).astype(jnp.int32)
    out = _sc_embed(ids, tts, word_emb, pos_emb, type_emb, gamma, beta)
    return out.reshape(B, S, H)


# speedup vs baseline: 4.5808x; 1.5167x over previous
import functools

import jax
import jax.numpy as jnp
from jax import lax
from jax.experimental import pallas as pl
from jax.experimental.pallas import tpu as pltpu
from jax.experimental.pallas import tpu_sc as plsc

NC, NS, L = 2, 16, 16
NW = NC * NS
B, S, H = 4, 2048, 768
TOK = B * S
TPW = TOK // NW
C = 16
NCH = TPW // C
NJ = H // L
EPS = 1e-12


_DNUMS = lax.GatherDimensionNumbers(
    offset_dims=(), collapsed_slice_dims=(0,), start_index_map=(0,))


def _lane_broadcast(vec, t):
    idxv = jnp.full((L,), t, jnp.int32)
    return lax.gather(vec, idxv[:, None], _DNUMS, slice_sizes=(1,),
                      mode=lax.GatherScatterMode.PROMISE_IN_BOUNDS)


def _body(ids, tts, word, pos, typ, gamma, beta, out,
          idx_v, tt_v, tybuf, wbuf, pbuf, xbuf, obuf, g_v, b_v, sbuf, qbuf,
          sem_g, sem_o):
    wid = lax.axis_index("s") * NC + lax.axis_index("c")
    base = wid * TPW
    s_base = lax.rem(base, S)

    pltpu.sync_copy(gamma, g_v)
    pltpu.sync_copy(beta, b_v)
    pltpu.sync_copy(typ, tybuf)
    pltpu.sync_copy(ids.at[pl.ds(base, TPW)], idx_v)
    pltpu.sync_copy(tts.at[pl.ds(base, TPW)], tt_v)

    def issue(cc, b):
        s0 = s_base + cc * C
        pltpu.async_copy(word.at[idx_v.at[pl.ds(cc * C, C)]], wbuf[b], sem_g[b])
        pltpu.async_copy(pos.at[pl.ds(s0, C)], pbuf[b], sem_g[b])

    def wait_gathers(b):
        pltpu.make_async_copy(word.at[idx_v.at[pl.ds(0, C)]], wbuf[b],
                              sem_g[b]).wait()
        pltpu.make_async_copy(pos.at[pl.ds(0, C)], pbuf[b], sem_g[b]).wait()

    def wait_scatter(cc, b):
        tok0 = base + cc * C
        pltpu.make_async_copy(obuf[b], out.at[pl.ds(tok0, C)], sem_o[b]).wait()

    def compute(cc, b):
        w, p, x_, o = wbuf[b], pbuf[b], xbuf[b], obuf[b]
        z = jnp.zeros((L,), jnp.float32)

        ttv = tt_v[pl.ds(cc * C, L)]
        masks = [_lane_broadcast(ttv, t) > 0 for t in range(C)]

        @plsc.parallel_loop(0, NJ, step=1, carry=(z,) * (2 * C))
        def p1(j, acc):
            acc = list(acc)
            sl = pl.ds(j * L, L)
            ty0 = tybuf[0, sl]
            ty1 = tybuf[1, sl]
            for t in range(C):
                x = w[t, sl] + p[t, sl] + jnp.where(masks[t], ty1, ty0)
                x_[t, sl] = x
                acc[t] = acc[t] + x
                acc[C + t] = acc[C + t] + x * x
            return tuple(acc)

        for t in range(C):
            sbuf[t, :] = p1[t]
            qbuf[t, :] = p1[C + t]
        rows = lax.iota(jnp.int32, L)
        tot_s = z
        tot_q = z
        for l in range(L):
            col = jnp.full((L,), l, jnp.int32)
            tot_s = tot_s + plsc.load_gather(sbuf, [rows, col])
            tot_q = tot_q + plsc.load_gather(qbuf, [rows, col])
        means = tot_s * (1.0 / H)
        varis = tot_q * (1.0 / H) - means * means
        v = varis + EPS
        i = lax.bitcast_convert_type(v, jnp.int32)
        i = 0x5F3759DF - lax.shift_right_logical(i, 1)
        r = lax.bitcast_convert_type(i, jnp.float32)
        for _ in range(4):
            r = r * (1.5 - 0.5 * v * r * r)
        mvs = [_lane_broadcast(means, t) for t in range(C)]
        rvs = [_lane_broadcast(r, t) for t in range(C)]

        @plsc.parallel_loop(0, NJ, step=1)
        def p2(j):
            sl = pl.ds(j * L, L)
            g = g_v[sl]
            bb = b_v[sl]
            for t in range(C):
                o[t, sl] = (x_[t, sl] - mvs[t]) * rvs[t] * g + bb

    issue(0, 0)
    issue(1, 1)

    def pair_body(i, carry):
        for b in range(2):
            cc = 2 * i + b
            wait_gathers(b)

            @pl.when(cc >= 2)
            def _():
                wait_scatter(cc - 2, b)

            compute(cc, b)
            tok0 = base + cc * C
            pltpu.async_copy(obuf[b], out.at[pl.ds(tok0, C)], sem_o[b])

            @pl.when(cc + 2 < NCH)
            def _():
                issue(cc + 2, b)
        return carry

    lax.fori_loop(0, NCH // 2, pair_body, 0)
    wait_scatter(NCH - 2, 0)
    wait_scatter(NCH - 1, 1)


_sc_embed = functools.partial(
    pl.kernel,
    mesh=plsc.VectorSubcoreMesh(core_axis_name="c", subcore_axis_name="s"),
    out_type=jax.ShapeDtypeStruct((TOK, H), jnp.float32),
    scratch_types=[
        pltpu.VMEM((TPW,), jnp.int32),
        pltpu.VMEM((TPW,), jnp.int32),
        pltpu.VMEM((2, H), jnp.float32),
        [pltpu.VMEM((C, H), jnp.float32) for _ in range(2)],
        [pltpu.VMEM((C, H), jnp.float32) for _ in range(2)],
        [pltpu.VMEM((C, H), jnp.float32) for _ in range(2)],
        [pltpu.VMEM((C, H), jnp.float32) for _ in range(2)],
        pltpu.VMEM((H,), jnp.float32),
        pltpu.VMEM((H,), jnp.float32),
        pltpu.VMEM((C, L), jnp.float32),
        pltpu.VMEM((C, L), jnp.float32),
        [pltpu.SemaphoreType.DMA for _ in range(2)],
        [pltpu.SemaphoreType.DMA for _ in range(2)],
    ],
    compiler_params=pltpu.CompilerParams(needs_layout_passes=False),
)(_body)


@jax.jit
def kernel(input_ids, token_type_ids, word_emb, pos_emb, type_emb, gamma, beta):
    ids = input_ids.reshape(-1).astype(jnp.int32)
    tts = token_type_ids.reshape(-1).astype(jnp.int32)
    out = _sc_embed(ids, tts, word_emb, pos_emb, type_emb, gamma, beta)
    return out.reshape(B, S, H)
